# baseline (speedup 1.0000x reference)
import jax
import jax.numpy as jnp
from jax import lax
from jax.experimental import pallas as pl
from jax.experimental.pallas import tpu as pltpu
from jax.experimental.pallas import tpu_sc as plsc

B = 16384
D = 768
N_MODALITY = 4
L = 16
NW = 32
NSLICE = 2
BS = B // NSLICE
B_PER_W = BS // NW
CH = 64
NCHUNK = B_PER_W // CH
RB = 4096
NBLK = BS // RB


def _gather_body(idx_hbm, qtab_hbm, out_hbm,
                 idx_v, q0, q1, qsem0, qsem1, ssem0, ssem1):
    wid = lax.axis_index("s") * 2 + lax.axis_index("c")
    wbase = wid * B_PER_W

    qb = [q0, q1]
    qsem = [qsem0, qsem1]
    ssem = [ssem0, ssem1]
    qcp = [None, None]
    scp = [None, None]

    pltpu.sync_copy(idx_hbm.at[pl.ds(wbase, B_PER_W)], idx_v)

    def start(c):
        b = c % 2
        qcp[b] = pltpu.async_copy(
            qtab_hbm.at[idx_v.at[pl.ds(c * CH, CH)]], qb[b], qsem[b])

    def process(c):
        b = c % 2
        qcp[b].wait()
        scp[b] = pltpu.async_copy(
            qb[b], out_hbm.at[pl.ds(wbase + c * CH, CH)], ssem[b])

    start(0)
    for c in range(NCHUNK):
        if c + 1 < NCHUNK:
            if c >= 1:
                scp[(c + 1) % 2].wait()
            start(c + 1)
        process(c)
    scp[0].wait()
    scp[1].wait()


def _add_body(code_ref, mtab_ref, g_ref, o_ref):
    code = code_ref[0, 0, :]
    onehot = (code[:, None]
              == lax.broadcasted_iota(jnp.int32, (RB, N_MODALITY), 1)
              ).astype(jnp.float32)
    mod = jnp.dot(onehot, mtab_ref[...], preferred_element_type=jnp.float32)
    o_ref[...] = g_ref[...] + mod


@jax.jit
def _run(idx, code, qtab, mtab):
    mesh = plsc.VectorSubcoreMesh(core_axis_name="c", subcore_axis_name="s")
    sc_gather = pl.kernel(
        _gather_body,
        out_type=jax.ShapeDtypeStruct((BS, D), jnp.float32),
        mesh=mesh,
        scratch_types=[
            pltpu.VMEM((B_PER_W,), jnp.int32),
            pltpu.VMEM((CH, D), jnp.float32),
            pltpu.VMEM((CH, D), jnp.float32),
            pltpu.SemaphoreType.DMA,
            pltpu.SemaphoreType.DMA,
            pltpu.SemaphoreType.DMA,
            pltpu.SemaphoreType.DMA,
        ],
    )

    tc_add = pl.pallas_call(
        _add_body,
        out_shape=jax.ShapeDtypeStruct((BS, D), jnp.float32),
        grid=(NBLK,),
        in_specs=[
            pl.BlockSpec((1, 1, RB), lambda i: (i, 0, 0)),
            pl.BlockSpec((N_MODALITY, D), lambda i: (0, 0)),
            pl.BlockSpec((RB, D), lambda i: (i, 0)),
        ],
        out_specs=pl.BlockSpec((RB, D), lambda i: (i, 0)),
        input_output_aliases={2: 0},
    )

    gs = [sc_gather(idx[s * BS:(s + 1) * BS], qtab) for s in range(NSLICE)]
    outs = [
        tc_add(code[s * BS:(s + 1) * BS].reshape(NBLK, 1, RB), mtab, gs[s])
        for s in range(NSLICE)
    ]
    return jnp.concatenate(outs, axis=0)


def kernel(input_text, modality_code, query_table, modality_table):
    idx = input_text.astype(jnp.int32)
    code = modality_code.astype(jnp.int32)
    return _run(idx, code, query_table, modality_table)


# speedup vs baseline: 1.3420x; 1.3420x over previous
import jax
import jax.numpy as jnp
from jax import lax
from jax.experimental import pallas as pl
from jax.experimental.pallas import tpu as pltpu
from jax.experimental.pallas import tpu_sc as plsc

B = 16384
D = 768
N_MODALITY = 4
L = 16
NW = 32
B_PER_W = B // NW
CH = 64
NCHUNK = B_PER_W // CH
RB = 4096
NBLK = B // RB


def _gather_body(idx_hbm, qtab_hbm, out_hbm,
                 idx_v, q0, q1, qsem0, qsem1, ssem0, ssem1):
    wid = lax.axis_index("s") * 2 + lax.axis_index("c")
    wbase = wid * B_PER_W

    qb = [q0, q1]
    qsem = [qsem0, qsem1]
    ssem = [ssem0, ssem1]
    qcp = [None, None]
    scp = [None, None]

    pltpu.sync_copy(idx_hbm.at[pl.ds(wbase, B_PER_W)], idx_v)

    def start(c):
        b = c % 2
        qcp[b] = pltpu.async_copy(
            qtab_hbm.at[idx_v.at[pl.ds(c * CH, CH)]], qb[b], qsem[b])

    def process(c):
        b = c % 2
        qcp[b].wait()
        scp[b] = pltpu.async_copy(
            qb[b], out_hbm.at[pl.ds(wbase + c * CH, CH)], ssem[b])

    start(0)
    for c in range(NCHUNK):
        if c + 1 < NCHUNK:
            if c >= 1:
                scp[(c + 1) % 2].wait()
            start(c + 1)
        process(c)
    scp[0].wait()
    scp[1].wait()


def _add_body(code_ref, mtab_ref, g_ref, o_ref):
    code = code_ref[0, 0, :]
    onehot = (code[:, None]
              == lax.broadcasted_iota(jnp.int32, (RB, N_MODALITY), 1)
              ).astype(jnp.float32)
    mod = jnp.dot(onehot, mtab_ref[...], preferred_element_type=jnp.float32)
    o_ref[...] = g_ref[...] + mod


@jax.jit
def _run(idx, code, qtab, mtab):
    mesh = plsc.VectorSubcoreMesh(core_axis_name="c", subcore_axis_name="s")
    gathered = pl.kernel(
        _gather_body,
        out_type=jax.ShapeDtypeStruct((B, D), jnp.float32),
        mesh=mesh,
        scratch_types=[
            pltpu.VMEM((B_PER_W,), jnp.int32),
            pltpu.VMEM((CH, D), jnp.float32),
            pltpu.VMEM((CH, D), jnp.float32),
            pltpu.SemaphoreType.DMA,
            pltpu.SemaphoreType.DMA,
            pltpu.SemaphoreType.DMA,
            pltpu.SemaphoreType.DMA,
        ],
    )(idx, qtab)

    code3 = code.reshape(NBLK, 1, RB)
    return pl.pallas_call(
        _add_body,
        out_shape=jax.ShapeDtypeStruct((B, D), jnp.float32),
        grid=(NBLK,),
        in_specs=[
            pl.BlockSpec((1, 1, RB), lambda i: (i, 0, 0)),
            pl.BlockSpec((N_MODALITY, D), lambda i: (0, 0)),
            pl.BlockSpec((RB, D), lambda i: (i, 0)),
        ],
        out_specs=pl.BlockSpec((RB, D), lambda i: (i, 0)),
    )(code3, mtab, gathered)


def kernel(input_text, modality_code, query_table, modality_table):
    idx = input_text.astype(jnp.int32)
    code = modality_code.astype(jnp.int32)
    return _run(idx, code, query_table, modality_table)
